# baseline (speedup 1.0000x reference)
import functools

import jax
import jax.numpy as jnp
import numpy as np
from jax import lax
from jax.experimental import pallas as pl
from jax.experimental.pallas import tpu as pltpu
from jax.experimental.pallas import tpu_sc as plsc

N_CLASS = 1000000
DROPOUT_PROB = 0.1


def _drop_mask(n: int):
    u = jax.random.uniform(jax.random.key(1234), (n,))
    return (u < DROPOUT_PROB).astype(jnp.int32)


@functools.lru_cache
def _build(batch: int, hidden: int):
    info = plsc.get_sparse_core_info()
    nc, ns, lanes = info.num_cores, info.num_subcores, info.num_lanes
    nw = nc * ns
    assert batch % (8 * nw) == 0 and hidden % lanes == 0
    b_per_w = batch // nw
    mesh = plsc.VectorSubcoreMesh(core_axis_name="c", subcore_axis_name="s")

    @functools.partial(
        pl.kernel,
        mesh=mesh,
        out_type=jax.ShapeDtypeStruct((batch, hidden), jnp.float32),
        compiler_params=pltpu.CompilerParams(use_tc_tiling_on_sc=False),
        scratch_types=[
            pltpu.VMEM((b_per_w,), jnp.int32),
            pltpu.VMEM((b_per_w,), jnp.int32),
            pltpu.VMEM((b_per_w,), jnp.int32),
            pltpu.VMEM((b_per_w, hidden), jnp.float32),
            pltpu.SemaphoreType.DMA,
        ],
    )
    def emb(table_hbm, labels_hbm, mask_hbm, out_hbm, lab_v, msk_v, idx_v, rows_v, sem):
        wid = lax.axis_index("s") * nc + lax.axis_index("c")
        base = wid * b_per_w
        pltpu.sync_copy(labels_hbm.at[pl.ds(base, b_per_w)], lab_v)
        pltpu.sync_copy(mask_hbm.at[pl.ds(base, b_per_w)], msk_v)
        for i in range(b_per_w // lanes):
            s = pl.ds(i * lanes, lanes)
            idx_v[s] = jnp.where(msk_v[s] != 0, N_CLASS, lab_v[s])
        pltpu.async_copy(table_hbm.at[idx_v], rows_v, sem).wait()
        pltpu.sync_copy(rows_v, out_hbm.at[pl.ds(base, b_per_w)])

    return emb


def kernel(labels, table):
    batch = labels.shape[0]
    hidden = table.shape[1]
    mask = _drop_mask(batch)
    emb = _build(batch, hidden)
    return emb(table, labels.astype(jnp.int32), jnp.asarray(mask))
